# Initial kernel scaffold; baseline (speedup 1.0000x reference)
import jax
import jax.numpy as jnp
from jax.experimental import pallas as pl


def kernel(x0, adj_s1_00_idx, adj_s1_00_val, adj_s1_10_idx, adj_s1_10_val, adj_s2_01_idx, adj_s2_01_val, adj_s2_10_idx, adj_s2_10_val, edges, rt_k, W_s1_l1_00, W_s1_l1_10, W_s1_l2_00, W_s1_l2_10, W_s2_l1_01, W_s2_l1_10, W_s2_l2_01, W_s2_l2_10, R_dec, D_dec):
    raise NotImplementedError("write your pallas kernel here")


# speedup vs baseline: 3.8645x; 3.8645x over previous
import functools

import jax
import jax.numpy as jnp
from jax import lax
from jax.experimental import pallas as pl
from jax.experimental.pallas import tpu as pltpu
from jax.experimental.pallas import tpu_sc as plsc

N = 50000
NPAD = 51200
DIN = 128
HH = 64
HC = 32
E = 800000
B = 4096
CHUNK = 128
NC, NS = 2, 16
NW = NC * NS
EPAD = 819200
IROWS = EPAD // CHUNK
GRP = 8
TCH = IROWS // NS
TGR = TCH // GRP
RPT = NPAD // NS
ZR = 640
BM = 3200
MB = NPAD // BM

_SC_PARAMS = pltpu.CompilerParams(use_tc_tiling_on_sc=False,
                                  needs_layout_passes=False)


def _sc_gcn_body(tab, src, dst, val, out, acc, idx_s, idx_d, val_v, rows_v,
                 zv, sem):
    c = lax.axis_index("c")
    s = lax.axis_index("s")
    cn = c * NPAD

    def _zrow(i, carry):
        zv[i, pl.ds(0, 16)] = jnp.zeros((16,), jnp.float32)
        zv[i, pl.ds(16, 16)] = jnp.zeros((16,), jnp.float32)
        return carry

    lax.fori_loop(0, ZR, _zrow, 0)
    for r in range(RPT // ZR):
        pltpu.sync_copy(zv, acc.at[pl.ds(s * RPT + r * ZR, ZR)])
    plsc.subcore_barrier()

    def _group(g, carry):
        row0 = s * TCH + g * GRP
        pltpu.sync_copy(src.at[pl.ds(row0, GRP)], idx_s)
        pltpu.sync_copy(dst.at[pl.ds(row0, GRP)], idx_d)
        pltpu.sync_copy(val.at[pl.ds(row0, GRP)], val_v)
        for j in range(GRP):
            for k in range(CHUNK // 16):
                sl = (j, pl.ds(k * 16, 16))
                idx_s[sl] = idx_s[sl] + cn

        def _chunk(j, inner):
            pltpu.async_copy(tab.at[idx_s.at[j]], rows_v, sem).wait()
            for k in range(CHUNK // 16):
                for e in range(16):
                    ee = k * 16 + e
                    sp = plsc.load_gather(
                        val_v,
                        [jnp.full((16,), j, jnp.int32),
                         jnp.full((16,), k * 16 + e, jnp.int32)])
                    rows_v[ee, pl.ds(0, 16)] = rows_v[ee, pl.ds(0, 16)] * sp
                    rows_v[ee, pl.ds(16, 16)] = rows_v[ee, pl.ds(16, 16)] * sp
            pltpu.sync_copy(rows_v, acc.at[idx_d.at[j]], add=True)
            return inner

        lax.fori_loop(0, GRP, _chunk, 0)
        return carry

    lax.fori_loop(0, TGR, _group, 0)
    plsc.subcore_barrier()
    pltpu.sync_copy(acc.at[pl.ds(s * RPT, RPT)],
                    out.at[pl.ds(cn + s * RPT, RPT)])


def _sc_gcn(tab, src, dst, val):
    f = pl.kernel(
        _sc_gcn_body,
        out_type=jax.ShapeDtypeStruct((2 * NPAD, HC), jnp.float32),
        mesh=plsc.VectorSubcoreMesh(core_axis_name="c", subcore_axis_name="s"),
        scratch_types=[
            pltpu.VMEM_SHARED((NPAD, HC), jnp.float32),
            pltpu.VMEM((GRP, CHUNK), jnp.int32),
            pltpu.VMEM((GRP, CHUNK), jnp.int32),
            pltpu.VMEM((GRP, CHUNK), jnp.float32),
            pltpu.VMEM((CHUNK, HC), jnp.float32),
            pltpu.VMEM((ZR, HC), jnp.float32),
            pltpu.SemaphoreType.DMA,
        ],
        compiler_params=_SC_PARAMS,
    )
    return f(tab, src, dst, val)


def _sc_dec_body(g20, g21, e0, e1, rlo, rhi, clo, chi, idx_v, rows_v, sem):
    c = lax.axis_index("c")
    s = lax.axis_index("s")
    w = s * NC + c
    base = w * CHUNK

    def _bump(delta):
        for k in range(CHUNK // 16):
            sl = pl.ds(k * 16, 16)
            idx_v[sl] = idx_v[sl] + delta

    pltpu.sync_copy(e0.at[pl.ds(base, CHUNK)], idx_v)
    pltpu.async_copy(g20.at[idx_v], rows_v, sem).wait()
    pltpu.sync_copy(rows_v, rlo.at[pl.ds(base, CHUNK)])
    _bump(NPAD)
    pltpu.async_copy(g20.at[idx_v], rows_v, sem).wait()
    pltpu.sync_copy(rows_v, rhi.at[pl.ds(base, CHUNK)])

    pltpu.sync_copy(e1.at[pl.ds(base, CHUNK)], idx_v)
    pltpu.async_copy(g21.at[idx_v], rows_v, sem).wait()
    pltpu.sync_copy(rows_v, clo.at[pl.ds(base, CHUNK)])
    _bump(NPAD)
    pltpu.async_copy(g21.at[idx_v], rows_v, sem).wait()
    pltpu.sync_copy(rows_v, chi.at[pl.ds(base, CHUNK)])


def _sc_dec(g20, g21, e0, e1):
    o = jax.ShapeDtypeStruct((B, HC), jnp.float32)
    f = pl.kernel(
        _sc_dec_body,
        out_type=(o, o, o, o),
        mesh=plsc.VectorSubcoreMesh(core_axis_name="c", subcore_axis_name="s"),
        scratch_types=[
            pltpu.VMEM((CHUNK,), jnp.int32),
            pltpu.VMEM((CHUNK, HC), jnp.float32),
            pltpu.SemaphoreType.DMA,
        ],
        compiler_params=_SC_PARAMS,
    )
    return f(g20, g21, e0, e1)


def _wh(w_ref):
    h = pl.program_id(1).astype(jnp.float32)
    w = w_ref[...]
    return w[:, :HC] * (1.0 - h) + w[:, HC:] * h


def _mm1_body(x_ref, w_ref, o_ref):
    o_ref[...] = jnp.dot(x_ref[...], _wh(w_ref),
                         preferred_element_type=jnp.float32)


def _mm1(x, w):
    return pl.pallas_call(
        _mm1_body,
        grid=(MB, 2),
        in_specs=[
            pl.BlockSpec((BM, DIN), lambda i, h: (i, 0)),
            pl.BlockSpec((DIN, HH), lambda i, h: (0, 0)),
        ],
        out_specs=pl.BlockSpec((BM, HC), lambda i, h: (h * MB + i, 0)),
        out_shape=jax.ShapeDtypeStruct((2 * NPAD, HC), jnp.float32),
    )(x, w)


def _mm2_body(xlo_ref, xhi_ref, w_ref, o_ref):
    x = jnp.concatenate(
        [jnp.maximum(xlo_ref[...], 0.0), jnp.maximum(xhi_ref[...], 0.0)],
        axis=1)
    o_ref[...] = jnp.dot(x, _wh(w_ref), preferred_element_type=jnp.float32)


def _mm2(hraw, w):
    return pl.pallas_call(
        _mm2_body,
        grid=(MB, 2),
        in_specs=[
            pl.BlockSpec((BM, HC), lambda i, h: (i, 0)),
            pl.BlockSpec((BM, HC), lambda i, h: (MB + i, 0)),
            pl.BlockSpec((HH, HH), lambda i, h: (0, 0)),
        ],
        out_specs=pl.BlockSpec((BM, HC), lambda i, h: (h * MB + i, 0)),
        out_shape=jax.ShapeDtypeStruct((2 * NPAD, HC), jnp.float32),
    )(hraw, hraw, w)


def _dec_body(rlo, rhi, clo, chi, r_ref, d_ref, o_ref):
    d = d_ref[...]
    rows = jnp.concatenate(
        [jnp.maximum(rlo[...], 0.0), jnp.maximum(rhi[...], 0.0)], axis=1) * d
    cols = jnp.concatenate(
        [jnp.maximum(clo[...], 0.0), jnp.maximum(chi[...], 0.0)], axis=1) * d
    t = jnp.dot(rows, r_ref[...], preferred_element_type=jnp.float32)
    o_ref[...] = jnp.sum(t * cols, axis=1)[None, :]


def _dec(rlo, rhi, clo, chi, r, d):
    spec = pl.BlockSpec((B, HC), lambda: (0, 0))
    return pl.pallas_call(
        _dec_body,
        in_specs=[spec, spec, spec,
                  spec,
                  pl.BlockSpec((HH, HH), lambda: (0, 0)),
                  pl.BlockSpec((1, HH), lambda: (0, 0))],
        out_specs=pl.BlockSpec((1, B), lambda: (0, 0)),
        out_shape=jax.ShapeDtypeStruct((1, B), jnp.float32),
    )(rlo, rhi, clo, chi, r, d)


def _prep(idx, val):
    pad = EPAD - E
    padi = jnp.arange(pad, dtype=jnp.int32)
    src = jnp.concatenate([idx[0], padi]).reshape(IROWS, CHUNK)
    dst = jnp.concatenate([idx[1], padi]).reshape(IROWS, CHUNK)
    vals = jnp.concatenate([val, jnp.zeros((pad,), val.dtype)])
    return src, dst, vals.reshape(IROWS, CHUNK)


def kernel(x0, adj_s1_00_idx, adj_s1_00_val, adj_s1_10_idx, adj_s1_10_val,
           adj_s2_01_idx, adj_s2_01_val, adj_s2_10_idx, adj_s2_10_val,
           edges, rt_k,
           W_s1_l1_00, W_s1_l1_10, W_s1_l2_00, W_s1_l2_10,
           W_s2_l1_01, W_s2_l1_10, W_s2_l2_01, W_s2_l2_10,
           R_dec, D_dec):
    a00 = _prep(adj_s1_00_idx, adj_s1_00_val)
    a10 = _prep(adj_s1_10_idx, adj_s1_10_val)
    b01 = _prep(adj_s2_01_idx, adj_s2_01_val)
    b10 = _prep(adj_s2_10_idx, adj_s2_10_val)

    x0p = jnp.concatenate(
        [x0, jnp.zeros((NPAD - N, DIN), jnp.float32)], axis=0)
    h10 = _sc_gcn(_mm1(x0p, W_s1_l1_00), *a00)
    h20 = _sc_gcn(_mm2(h10, W_s1_l2_00), *a00)
    h21 = _sc_gcn(_mm2(h10, W_s1_l2_10), *a10)
    g10 = _sc_gcn(_mm2(h21, W_s2_l1_01), *b01)
    g11 = _sc_gcn(_mm2(h20, W_s2_l1_10), *b10)
    g20 = _sc_gcn(_mm2(g11, W_s2_l2_01), *b01)
    g21 = _sc_gcn(_mm2(g10, W_s2_l2_10), *b10)

    e0 = edges[:, 0]
    e1 = edges[:, 1]
    rlo, rhi, clo, chi = _sc_dec(g20, g21, e0, e1)
    d = lax.dynamic_index_in_dim(D_dec, rt_k, 0, keepdims=True)
    preds = _dec(rlo, rhi, clo, chi, R_dec, d)
    return preds.reshape(B)


# speedup vs baseline: 3.9051x; 1.0105x over previous
import functools

import jax
import jax.numpy as jnp
from jax import lax
from jax.experimental import pallas as pl
from jax.experimental.pallas import tpu as pltpu
from jax.experimental.pallas import tpu_sc as plsc

N = 50000
NPAD = 51200
DIN = 128
HH = 64
HC = 32
E = 800000
B = 4096
CHUNK = 128
NC, NS = 2, 16
NW = NC * NS
EPAD = 819200
IROWS = EPAD // CHUNK
GRP = 8
TCH = IROWS // NS
TGR = TCH // GRP
RPT = NPAD // NS
BM = 3200
MB = NPAD // BM

_SC_PARAMS = pltpu.CompilerParams(use_tc_tiling_on_sc=False,
                                  needs_layout_passes=False)


PH = 20
TGR4 = PH // 4
ZR = 100


def _sc_gcn_body(tab, src3, dst2, val2, out, acc, idx_s, idx_d, val_v,
                 r0, r1, r2, r3, zv,
                 g0, g1, g2, g3, s0, s1, s2, s3):
    rows = (r0, r1, r2, r3)
    gsem = (g0, g1, g2, g3)
    ssem = (s0, s1, s2, s3)
    c = lax.axis_index("c")
    s = lax.axis_index("s")

    def _zrow(i, carry):
        zv[i, pl.ds(0, 16)] = jnp.zeros((16,), jnp.float32)
        zv[i, pl.ds(16, 16)] = jnp.zeros((16,), jnp.float32)
        return carry

    lax.fori_loop(0, ZR, _zrow, 0)
    for r in range(RPT // ZR):
        pltpu.sync_copy(zv, acc.at[pl.ds(s * RPT + r * ZR, ZR)])
    plsc.subcore_barrier()

    def _fire_gather(t, b):
        pltpu.async_copy(tab.at[idx_s.at[t]], rows[b], gsem[b])

    def _fire_scatter(t, b):
        pltpu.async_copy(rows[b], acc.at[idx_d.at[t]], ssem[b], add=True)

    def _wait(sem, b):
        pltpu.make_async_copy(tab.at[pl.ds(0, CHUNK)], rows[b], sem).wait()

    def _phase(phase, pcarry):
        row0 = s * TCH + phase * PH
        pltpu.sync_copy(src3.at[c, pl.ds(row0, PH)], idx_s)
        pltpu.sync_copy(dst2.at[pl.ds(row0, PH)], idx_d)
        pltpu.sync_copy(val2.at[pl.ds(row0, PH)], val_v)
        _fire_gather(0, 0)
        _fire_gather(1, 1)

        def _grp(g, carry):
            for u in range(4):
                t = g * 4 + u
                y = (u + 2) % 4
                _wait(gsem[u], u)
                rx = rows[u]
                for ee in range(CHUNK):
                    sp = plsc.load_gather(
                        val_v,
                        [jnp.full((16,), t, jnp.int32),
                         jnp.full((16,), ee, jnp.int32)])
                    a = rx[ee, pl.ds(0, 16)]
                    b = rx[ee, pl.ds(16, 16)]
                    rx[ee, pl.ds(0, 16)] = a * sp
                    rx[ee, pl.ds(16, 16)] = b * sp
                _fire_scatter(t, u)
                if u < 2:
                    @pl.when(g > 0)
                    def _():
                        _wait(ssem[y], y)

                    _fire_gather(t + 2, y)
                else:
                    _wait(ssem[y], y)

                    @pl.when(g < TGR4 - 1)
                    def _():
                        _fire_gather(t + 2, y)
            return carry

        lax.fori_loop(0, TGR4, _grp, 0)
        _wait(ssem[2], 2)
        _wait(ssem[3], 3)
        return pcarry

    lax.fori_loop(0, TCH // PH, _phase, 0)
    plsc.subcore_barrier()
    pltpu.sync_copy(acc.at[pl.ds(s * RPT, RPT)],
                    out.at[pl.ds(c * NPAD + s * RPT, RPT)])


def _sc_gcn(tab, src3, dst2, val2):
    f = pl.kernel(
        _sc_gcn_body,
        out_type=jax.ShapeDtypeStruct((2 * NPAD, HC), jnp.float32),
        mesh=plsc.VectorSubcoreMesh(core_axis_name="c", subcore_axis_name="s"),
        scratch_types=[
            pltpu.VMEM_SHARED((NPAD, HC), jnp.float32),
            pltpu.VMEM((PH, CHUNK), jnp.int32),
            pltpu.VMEM((PH, CHUNK), jnp.int32),
            pltpu.VMEM((PH, CHUNK), jnp.float32),
            pltpu.VMEM((CHUNK, HC), jnp.float32),
            pltpu.VMEM((CHUNK, HC), jnp.float32),
            pltpu.VMEM((CHUNK, HC), jnp.float32),
            pltpu.VMEM((CHUNK, HC), jnp.float32),
            pltpu.VMEM((ZR, HC), jnp.float32),
            pltpu.SemaphoreType.DMA,
            pltpu.SemaphoreType.DMA,
            pltpu.SemaphoreType.DMA,
            pltpu.SemaphoreType.DMA,
            pltpu.SemaphoreType.DMA,
            pltpu.SemaphoreType.DMA,
            pltpu.SemaphoreType.DMA,
            pltpu.SemaphoreType.DMA,
        ],
        compiler_params=_SC_PARAMS,
    )
    return f(tab, src3, dst2, val2)


def _sc_dec_body(g20, g21, e0, e1, rlo, rhi, clo, chi, idx_v, rows_v, sem):
    c = lax.axis_index("c")
    s = lax.axis_index("s")
    w = s * NC + c
    base = w * CHUNK

    def _bump(delta):
        for k in range(CHUNK // 16):
            sl = pl.ds(k * 16, 16)
            idx_v[sl] = idx_v[sl] + delta

    pltpu.sync_copy(e0.at[pl.ds(base, CHUNK)], idx_v)
    pltpu.async_copy(g20.at[idx_v], rows_v, sem).wait()
    pltpu.sync_copy(rows_v, rlo.at[pl.ds(base, CHUNK)])
    _bump(NPAD)
    pltpu.async_copy(g20.at[idx_v], rows_v, sem).wait()
    pltpu.sync_copy(rows_v, rhi.at[pl.ds(base, CHUNK)])

    pltpu.sync_copy(e1.at[pl.ds(base, CHUNK)], idx_v)
    pltpu.async_copy(g21.at[idx_v], rows_v, sem).wait()
    pltpu.sync_copy(rows_v, clo.at[pl.ds(base, CHUNK)])
    _bump(NPAD)
    pltpu.async_copy(g21.at[idx_v], rows_v, sem).wait()
    pltpu.sync_copy(rows_v, chi.at[pl.ds(base, CHUNK)])


def _sc_dec(g20, g21, e0, e1):
    o = jax.ShapeDtypeStruct((B, HC), jnp.float32)
    f = pl.kernel(
        _sc_dec_body,
        out_type=(o, o, o, o),
        mesh=plsc.VectorSubcoreMesh(core_axis_name="c", subcore_axis_name="s"),
        scratch_types=[
            pltpu.VMEM((CHUNK,), jnp.int32),
            pltpu.VMEM((CHUNK, HC), jnp.float32),
            pltpu.SemaphoreType.DMA,
        ],
        compiler_params=_SC_PARAMS,
    )
    return f(g20, g21, e0, e1)


def _wh(w_ref):
    h = pl.program_id(1).astype(jnp.float32)
    w = w_ref[...]
    return w[:, :HC] * (1.0 - h) + w[:, HC:] * h


def _mm1_body(x_ref, w_ref, o_ref):
    o_ref[...] = jnp.dot(x_ref[...], _wh(w_ref),
                         preferred_element_type=jnp.float32)


def _mm1(x, w):
    return pl.pallas_call(
        _mm1_body,
        grid=(MB, 2),
        in_specs=[
            pl.BlockSpec((BM, DIN), lambda i, h: (i, 0)),
            pl.BlockSpec((DIN, HH), lambda i, h: (0, 0)),
        ],
        out_specs=pl.BlockSpec((BM, HC), lambda i, h: (h * MB + i, 0)),
        out_shape=jax.ShapeDtypeStruct((2 * NPAD, HC), jnp.float32),
    )(x, w)


def _mm2_body(xlo_ref, xhi_ref, w_ref, o_ref):
    x = jnp.concatenate(
        [jnp.maximum(xlo_ref[...], 0.0), jnp.maximum(xhi_ref[...], 0.0)],
        axis=1)
    o_ref[...] = jnp.dot(x, _wh(w_ref), preferred_element_type=jnp.float32)


def _mm2(hraw, w):
    return pl.pallas_call(
        _mm2_body,
        grid=(MB, 2),
        in_specs=[
            pl.BlockSpec((BM, HC), lambda i, h: (i, 0)),
            pl.BlockSpec((BM, HC), lambda i, h: (MB + i, 0)),
            pl.BlockSpec((HH, HH), lambda i, h: (0, 0)),
        ],
        out_specs=pl.BlockSpec((BM, HC), lambda i, h: (h * MB + i, 0)),
        out_shape=jax.ShapeDtypeStruct((2 * NPAD, HC), jnp.float32),
    )(hraw, hraw, w)


def _dec_body(rlo, rhi, clo, chi, r_ref, d_ref, o_ref):
    d = d_ref[...]
    rows = jnp.concatenate(
        [jnp.maximum(rlo[...], 0.0), jnp.maximum(rhi[...], 0.0)], axis=1) * d
    cols = jnp.concatenate(
        [jnp.maximum(clo[...], 0.0), jnp.maximum(chi[...], 0.0)], axis=1) * d
    t = jnp.dot(rows, r_ref[...], preferred_element_type=jnp.float32)
    o_ref[...] = jnp.sum(t * cols, axis=1)[None, :]


def _dec(rlo, rhi, clo, chi, r, d):
    spec = pl.BlockSpec((B, HC), lambda: (0, 0))
    return pl.pallas_call(
        _dec_body,
        in_specs=[spec, spec, spec,
                  spec,
                  pl.BlockSpec((HH, HH), lambda: (0, 0)),
                  pl.BlockSpec((1, HH), lambda: (0, 0))],
        out_specs=pl.BlockSpec((1, B), lambda: (0, 0)),
        out_shape=jax.ShapeDtypeStruct((1, B), jnp.float32),
    )(rlo, rhi, clo, chi, r, d)


def _prep(idx, val):
    pad = EPAD - E
    padi = jnp.arange(pad, dtype=jnp.int32)
    src = jnp.concatenate([idx[0], padi]).reshape(IROWS, CHUNK)
    src3 = jnp.stack([src, src + NPAD])
    dst = jnp.concatenate([idx[1], padi]).reshape(IROWS, CHUNK)
    vals = jnp.concatenate([val, jnp.zeros((pad,), val.dtype)])
    return src3, dst, vals.reshape(IROWS, CHUNK)


def kernel(x0, adj_s1_00_idx, adj_s1_00_val, adj_s1_10_idx, adj_s1_10_val,
           adj_s2_01_idx, adj_s2_01_val, adj_s2_10_idx, adj_s2_10_val,
           edges, rt_k,
           W_s1_l1_00, W_s1_l1_10, W_s1_l2_00, W_s1_l2_10,
           W_s2_l1_01, W_s2_l1_10, W_s2_l2_01, W_s2_l2_10,
           R_dec, D_dec):
    a00 = _prep(adj_s1_00_idx, adj_s1_00_val)
    a10 = _prep(adj_s1_10_idx, adj_s1_10_val)
    b01 = _prep(adj_s2_01_idx, adj_s2_01_val)
    b10 = _prep(adj_s2_10_idx, adj_s2_10_val)

    x0p = jnp.concatenate(
        [x0, jnp.zeros((NPAD - N, DIN), jnp.float32)], axis=0)
    h10 = _sc_gcn(_mm1(x0p, W_s1_l1_00), *a00)
    h20 = _sc_gcn(_mm2(h10, W_s1_l2_00), *a00)
    h21 = _sc_gcn(_mm2(h10, W_s1_l2_10), *a10)
    g10 = _sc_gcn(_mm2(h21, W_s2_l1_01), *b01)
    g11 = _sc_gcn(_mm2(h20, W_s2_l1_10), *b10)
    g20 = _sc_gcn(_mm2(g11, W_s2_l2_01), *b01)
    g21 = _sc_gcn(_mm2(g10, W_s2_l2_10), *b10)

    e0 = edges[:, 0]
    e1 = edges[:, 1]
    rlo, rhi, clo, chi = _sc_dec(g20, g21, e0, e1)
    d = lax.dynamic_index_in_dim(D_dec, rt_k, 0, keepdims=True)
    preds = _dec(rlo, rhi, clo, chi, R_dec, d)
    return preds.reshape(B)


# speedup vs baseline: 9.0216x; 2.3102x over previous
import functools

import jax
import jax.numpy as jnp
from jax import lax
from jax.experimental import pallas as pl
from jax.experimental.pallas import tpu as pltpu
from jax.experimental.pallas import tpu_sc as plsc

N = 50000
NPAD = 51200
DIN = 128
HH = 64
HC = 32
E = 800000
B = 4096
CHUNK = 128
NC, NS = 2, 16
NW = NC * NS
EPAD = 819200
IROWS = EPAD // CHUNK
GRP = 8
TCH = IROWS // NS
TGR = TCH // GRP
RPT = NPAD // NS
BM = 3200
MB = NPAD // BM

_SC_PARAMS = pltpu.CompilerParams(use_tc_tiling_on_sc=False,
                                  needs_layout_passes=False)


PH = 20
TGR4 = PH // 4
ZR = 100


_GDN = lax.GatherDimensionNumbers(
    offset_dims=(), collapsed_slice_dims=(0,), start_index_map=(0,))


def _lane_splat(v16, e):
    idx = jnp.full((16, 1), e, jnp.int32)
    return lax.gather(v16, idx, _GDN, (1,),
                      mode=lax.GatherScatterMode.PROMISE_IN_BOUNDS)


def _sc_gcn_body(tab, src3, dst2, val2, out, acc, idx_s, idx_d, val_v,
                 r0, r1, r2, r3, zv,
                 g0, g1, g2, g3, s0, s1, s2, s3):
    rows = (r0, r1, r2, r3)
    gsem = (g0, g1, g2, g3)
    ssem = (s0, s1, s2, s3)
    c = lax.axis_index("c")
    s = lax.axis_index("s")

    def _zrow(i, carry):
        zv[i, pl.ds(0, 16)] = jnp.zeros((16,), jnp.float32)
        zv[i, pl.ds(16, 16)] = jnp.zeros((16,), jnp.float32)
        return carry

    lax.fori_loop(0, ZR, _zrow, 0)
    for r in range(RPT // ZR):
        pltpu.sync_copy(zv, acc.at[pl.ds(s * RPT + r * ZR, ZR)])
    plsc.subcore_barrier()

    def _fire_gather(t, b):
        pltpu.async_copy(tab.at[idx_s.at[t]], rows[b], gsem[b])

    def _fire_scatter(t, b):
        pltpu.async_copy(rows[b], acc.at[idx_d.at[t]], ssem[b], add=True)

    def _wait(sem, b):
        pltpu.make_async_copy(tab.at[pl.ds(0, CHUNK)], rows[b], sem).wait()

    def _phase(phase, pcarry):
        row0 = s * TCH + phase * PH
        pltpu.sync_copy(src3.at[c, pl.ds(row0, PH)], idx_s)
        pltpu.sync_copy(dst2.at[pl.ds(row0, PH)], idx_d)
        pltpu.sync_copy(val2.at[pl.ds(row0, PH)], val_v)
        _fire_gather(0, 0)
        _fire_gather(1, 1)

        def _grp(g, carry):
            for u in range(4):
                t = g * 4 + u
                y = (u + 2) % 4
                _wait(gsem[u], u)
                rx = rows[u]
                for k in range(CHUNK // 16):
                    v16 = val_v[t, pl.ds(k * 16, 16)]
                    for e in range(16):
                        ee = k * 16 + e
                        sp = _lane_splat(v16, e)
                        a = rx[ee, pl.ds(0, 16)]
                        b = rx[ee, pl.ds(16, 16)]
                        rx[ee, pl.ds(0, 16)] = a * sp
                        rx[ee, pl.ds(16, 16)] = b * sp
                _fire_scatter(t, u)
                if u < 2:
                    @pl.when(g > 0)
                    def _():
                        _wait(ssem[y], y)

                    _fire_gather(t + 2, y)
                else:
                    _wait(ssem[y], y)

                    @pl.when(g < TGR4 - 1)
                    def _():
                        _fire_gather(t + 2, y)
            return carry

        lax.fori_loop(0, TGR4, _grp, 0)
        _wait(ssem[2], 2)
        _wait(ssem[3], 3)
        return pcarry

    lax.fori_loop(0, TCH // PH, _phase, 0)
    plsc.subcore_barrier()
    pltpu.sync_copy(acc.at[pl.ds(s * RPT, RPT)],
                    out.at[pl.ds(c * NPAD + s * RPT, RPT)])


def _sc_gcn(tab, src3, dst2, val2):
    f = pl.kernel(
        _sc_gcn_body,
        out_type=jax.ShapeDtypeStruct((2 * NPAD, HC), jnp.float32),
        mesh=plsc.VectorSubcoreMesh(core_axis_name="c", subcore_axis_name="s"),
        scratch_types=[
            pltpu.VMEM_SHARED((NPAD, HC), jnp.float32),
            pltpu.VMEM((PH, CHUNK), jnp.int32),
            pltpu.VMEM((PH, CHUNK), jnp.int32),
            pltpu.VMEM((PH, CHUNK), jnp.float32),
            pltpu.VMEM((CHUNK, HC), jnp.float32),
            pltpu.VMEM((CHUNK, HC), jnp.float32),
            pltpu.VMEM((CHUNK, HC), jnp.float32),
            pltpu.VMEM((CHUNK, HC), jnp.float32),
            pltpu.VMEM((ZR, HC), jnp.float32),
            pltpu.SemaphoreType.DMA,
            pltpu.SemaphoreType.DMA,
            pltpu.SemaphoreType.DMA,
            pltpu.SemaphoreType.DMA,
            pltpu.SemaphoreType.DMA,
            pltpu.SemaphoreType.DMA,
            pltpu.SemaphoreType.DMA,
            pltpu.SemaphoreType.DMA,
        ],
        compiler_params=_SC_PARAMS,
    )
    return f(tab, src3, dst2, val2)


def _sc_dec_body(g20, g21, e0, e1, rlo, rhi, clo, chi, idx_v, rows_v, sem):
    c = lax.axis_index("c")
    s = lax.axis_index("s")
    w = s * NC + c
    base = w * CHUNK

    def _bump(delta):
        for k in range(CHUNK // 16):
            sl = pl.ds(k * 16, 16)
            idx_v[sl] = idx_v[sl] + delta

    pltpu.sync_copy(e0.at[pl.ds(base, CHUNK)], idx_v)
    pltpu.async_copy(g20.at[idx_v], rows_v, sem).wait()
    pltpu.sync_copy(rows_v, rlo.at[pl.ds(base, CHUNK)])
    _bump(NPAD)
    pltpu.async_copy(g20.at[idx_v], rows_v, sem).wait()
    pltpu.sync_copy(rows_v, rhi.at[pl.ds(base, CHUNK)])

    pltpu.sync_copy(e1.at[pl.ds(base, CHUNK)], idx_v)
    pltpu.async_copy(g21.at[idx_v], rows_v, sem).wait()
    pltpu.sync_copy(rows_v, clo.at[pl.ds(base, CHUNK)])
    _bump(NPAD)
    pltpu.async_copy(g21.at[idx_v], rows_v, sem).wait()
    pltpu.sync_copy(rows_v, chi.at[pl.ds(base, CHUNK)])


def _sc_dec(g20, g21, e0, e1):
    o = jax.ShapeDtypeStruct((B, HC), jnp.float32)
    f = pl.kernel(
        _sc_dec_body,
        out_type=(o, o, o, o),
        mesh=plsc.VectorSubcoreMesh(core_axis_name="c", subcore_axis_name="s"),
        scratch_types=[
            pltpu.VMEM((CHUNK,), jnp.int32),
            pltpu.VMEM((CHUNK, HC), jnp.float32),
            pltpu.SemaphoreType.DMA,
        ],
        compiler_params=_SC_PARAMS,
    )
    return f(g20, g21, e0, e1)


def _wh(w_ref):
    h = pl.program_id(1).astype(jnp.float32)
    w = w_ref[...]
    return w[:, :HC] * (1.0 - h) + w[:, HC:] * h


def _mm1_body(x_ref, w_ref, o_ref):
    o_ref[...] = jnp.dot(x_ref[...], _wh(w_ref),
                         preferred_element_type=jnp.float32)


def _mm1(x, w):
    return pl.pallas_call(
        _mm1_body,
        grid=(MB, 2),
        in_specs=[
            pl.BlockSpec((BM, DIN), lambda i, h: (i, 0)),
            pl.BlockSpec((DIN, HH), lambda i, h: (0, 0)),
        ],
        out_specs=pl.BlockSpec((BM, HC), lambda i, h: (h * MB + i, 0)),
        out_shape=jax.ShapeDtypeStruct((2 * NPAD, HC), jnp.float32),
    )(x, w)


def _mm2_body(xlo_ref, xhi_ref, w_ref, o_ref):
    x = jnp.concatenate(
        [jnp.maximum(xlo_ref[...], 0.0), jnp.maximum(xhi_ref[...], 0.0)],
        axis=1)
    o_ref[...] = jnp.dot(x, _wh(w_ref), preferred_element_type=jnp.float32)


def _mm2(hraw, w):
    return pl.pallas_call(
        _mm2_body,
        grid=(MB, 2),
        in_specs=[
            pl.BlockSpec((BM, HC), lambda i, h: (i, 0)),
            pl.BlockSpec((BM, HC), lambda i, h: (MB + i, 0)),
            pl.BlockSpec((HH, HH), lambda i, h: (0, 0)),
        ],
        out_specs=pl.BlockSpec((BM, HC), lambda i, h: (h * MB + i, 0)),
        out_shape=jax.ShapeDtypeStruct((2 * NPAD, HC), jnp.float32),
    )(hraw, hraw, w)


def _dec_body(rlo, rhi, clo, chi, r_ref, d_ref, o_ref):
    d = d_ref[...]
    rows = jnp.concatenate(
        [jnp.maximum(rlo[...], 0.0), jnp.maximum(rhi[...], 0.0)], axis=1) * d
    cols = jnp.concatenate(
        [jnp.maximum(clo[...], 0.0), jnp.maximum(chi[...], 0.0)], axis=1) * d
    t = jnp.dot(rows, r_ref[...], preferred_element_type=jnp.float32)
    o_ref[...] = jnp.sum(t * cols, axis=1)[None, :]


def _dec(rlo, rhi, clo, chi, r, d):
    spec = pl.BlockSpec((B, HC), lambda: (0, 0))
    return pl.pallas_call(
        _dec_body,
        in_specs=[spec, spec, spec,
                  spec,
                  pl.BlockSpec((HH, HH), lambda: (0, 0)),
                  pl.BlockSpec((1, HH), lambda: (0, 0))],
        out_specs=pl.BlockSpec((1, B), lambda: (0, 0)),
        out_shape=jax.ShapeDtypeStruct((1, B), jnp.float32),
    )(rlo, rhi, clo, chi, r, d)


def _prep(idx, val):
    pad = EPAD - E
    padi = jnp.arange(pad, dtype=jnp.int32)
    src = jnp.concatenate([idx[0], padi]).reshape(IROWS, CHUNK)
    src3 = jnp.stack([src, src + NPAD])
    dst = jnp.concatenate([idx[1], padi]).reshape(IROWS, CHUNK)
    vals = jnp.concatenate([val, jnp.zeros((pad,), val.dtype)])
    return src3, dst, vals.reshape(IROWS, CHUNK)


def kernel(x0, adj_s1_00_idx, adj_s1_00_val, adj_s1_10_idx, adj_s1_10_val,
           adj_s2_01_idx, adj_s2_01_val, adj_s2_10_idx, adj_s2_10_val,
           edges, rt_k,
           W_s1_l1_00, W_s1_l1_10, W_s1_l2_00, W_s1_l2_10,
           W_s2_l1_01, W_s2_l1_10, W_s2_l2_01, W_s2_l2_10,
           R_dec, D_dec):
    a00 = _prep(adj_s1_00_idx, adj_s1_00_val)
    a10 = _prep(adj_s1_10_idx, adj_s1_10_val)
    b01 = _prep(adj_s2_01_idx, adj_s2_01_val)
    b10 = _prep(adj_s2_10_idx, adj_s2_10_val)

    x0p = jnp.concatenate(
        [x0, jnp.zeros((NPAD - N, DIN), jnp.float32)], axis=0)
    h10 = _sc_gcn(_mm1(x0p, W_s1_l1_00), *a00)
    h20 = _sc_gcn(_mm2(h10, W_s1_l2_00), *a00)
    h21 = _sc_gcn(_mm2(h10, W_s1_l2_10), *a10)
    g10 = _sc_gcn(_mm2(h21, W_s2_l1_01), *b01)
    g11 = _sc_gcn(_mm2(h20, W_s2_l1_10), *b10)
    g20 = _sc_gcn(_mm2(g11, W_s2_l2_01), *b01)
    g21 = _sc_gcn(_mm2(g10, W_s2_l2_10), *b10)

    e0 = edges[:, 0]
    e1 = edges[:, 1]
    rlo, rhi, clo, chi = _sc_dec(g20, g21, e0, e1)
    d = lax.dynamic_index_in_dim(D_dec, rt_k, 0, keepdims=True)
    preds = _dec(rlo, rhi, clo, chi, R_dec, d)
    return preds.reshape(B)
